# baseline (speedup 1.0000x reference)
import functools

import jax
import jax.numpy as jnp
from jax import lax
from jax.experimental import pallas as pl
from jax.experimental.pallas import tpu as pltpu
from jax.experimental.pallas import tpu_sc as plsc

INPUT_DIM = 1000000
OUTPUT_DIM = 64
BATCH = 4096
SEQ_LEN = 200

N = BATCH * SEQ_LEN
NC, NS = 2, 16
NW = NC * NS
PER_W = N // NW
CHUNK = 128
NCHUNK = PER_W // CHUNK
NBUF = 8

_mesh = plsc.VectorSubcoreMesh(core_axis_name="c", subcore_axis_name="s")


@functools.partial(
    pl.kernel,
    out_type=jax.ShapeDtypeStruct((N, OUTPUT_DIM), jnp.float32),
    mesh=_mesh,
    scratch_types=[
        pltpu.VMEM((NCHUNK, CHUNK), jnp.int32),
        pltpu.VMEM((NBUF, CHUNK, OUTPUT_DIM), jnp.float32),
        pltpu.SemaphoreType.DMA((NBUF,)),
        pltpu.SemaphoreType.DMA((NBUF,)),
    ],
    compiler_params=pltpu.CompilerParams(use_tc_tiling_on_sc=False),
)
def _emb_lookup(x_hbm, w_hbm, out_hbm, idx_v, rows_v, in_sems, out_sems):
    wid = lax.axis_index("s") * NC + lax.axis_index("c")
    base = wid * PER_W

    pltpu.sync_copy(x_hbm.at[wid], idx_v)

    def start_gather(g, j):
        pltpu.async_copy(w_hbm.at[idx_v.at[g]], rows_v.at[j], in_sems.at[j])

    def wait_gather(g, j):
        pltpu.make_async_copy(
            w_hbm.at[idx_v.at[g]], rows_v.at[j], in_sems.at[j]
        ).wait()

    def start_write(g, j):
        pltpu.async_copy(
            rows_v.at[j],
            out_hbm.at[pl.ds(base + g * CHUNK, CHUNK)],
            out_sems.at[j],
        )

    def wait_write(g, j):
        pltpu.make_async_copy(
            rows_v.at[j],
            out_hbm.at[pl.ds(base + g * CHUNK, CHUNK)],
            out_sems.at[j],
        ).wait()

    for j in range(NBUF):
        start_gather(j, j)

    @pl.loop(0, NCHUNK - NBUF, step=NBUF)
    def _ring(g0):
        for j in range(NBUF):
            g = g0 + j
            wait_gather(g, j)
            start_write(g, j)
            wait_write(g, j)
            start_gather(g + NBUF, j)

    for j in range(NBUF):
        g = NCHUNK - NBUF + j
        wait_gather(g, j)
        start_write(g, j)
    for j in range(NBUF):
        g = NCHUNK - NBUF + j
        wait_write(g, j)


def kernel(x, W):
    xf = x.astype(jnp.int32).reshape(NW, NCHUNK, CHUNK)
    out = _emb_lookup(xf, W)
    return out.reshape(BATCH, SEQ_LEN, OUTPUT_DIM)


# speedup vs baseline: 1.0051x; 1.0051x over previous
import functools

import jax
import jax.numpy as jnp
from jax import lax
from jax.experimental import pallas as pl
from jax.experimental.pallas import tpu as pltpu
from jax.experimental.pallas import tpu_sc as plsc

INPUT_DIM = 1000000
OUTPUT_DIM = 64
BATCH = 4096
SEQ_LEN = 200

NC, NS = 2, 16
NW = NC * NS
CHUNK = BATCH // NW
NBUF = 8

_mesh = plsc.VectorSubcoreMesh(core_axis_name="c", subcore_axis_name="s")


@functools.partial(
    pl.kernel,
    out_type=jax.ShapeDtypeStruct((BATCH, SEQ_LEN, OUTPUT_DIM), jnp.float32),
    mesh=_mesh,
    scratch_types=[
        pltpu.VMEM((SEQ_LEN, CHUNK), jnp.int32),
        pltpu.VMEM((NBUF, CHUNK, OUTPUT_DIM), jnp.float32),
        pltpu.SemaphoreType.DMA((NBUF,)),
        pltpu.SemaphoreType.DMA((NBUF,)),
    ],
    compiler_params=pltpu.CompilerParams(use_tc_tiling_on_sc=False),
)
def _emb_lookup(xt_hbm, w_hbm, out_hbm, idx_v, rows_v, in_sems, out_sems):
    wid = lax.axis_index("s") * NC + lax.axis_index("c")
    b0 = wid * CHUNK

    pltpu.sync_copy(xt_hbm.at[:, pl.ds(b0, CHUNK)], idx_v)

    def start_gather(s, j):
        pltpu.async_copy(w_hbm.at[idx_v.at[s]], rows_v.at[j], in_sems.at[j])

    def wait_gather(s, j):
        pltpu.make_async_copy(
            w_hbm.at[idx_v.at[s]], rows_v.at[j], in_sems.at[j]
        ).wait()

    def start_write(s, j):
        pltpu.async_copy(
            rows_v.at[j], out_hbm.at[pl.ds(b0, CHUNK), s], out_sems.at[j]
        )

    def wait_write(s, j):
        pltpu.make_async_copy(
            rows_v.at[j], out_hbm.at[pl.ds(b0, CHUNK), s], out_sems.at[j]
        ).wait()

    for j in range(NBUF):
        start_gather(j, j)

    @pl.loop(0, SEQ_LEN - NBUF, step=NBUF)
    def _ring(s0):
        for j in range(NBUF):
            s = s0 + j
            wait_gather(s, j)
            start_write(s, j)
            wait_write(s, j)
            start_gather(s + NBUF, j)

    for j in range(NBUF):
        s = SEQ_LEN - NBUF + j
        wait_gather(s, j)
        start_write(s, j)
    for j in range(NBUF):
        s = SEQ_LEN - NBUF + j
        wait_write(s, j)


def kernel(x, W):
    xt = x.astype(jnp.int32).T
    return _emb_lookup(xt, W)
